# Initial kernel scaffold; baseline (speedup 1.0000x reference)
import functools

import jax
import jax.numpy as jnp
from jax.experimental import pallas as pl
from jax.experimental.pallas import tpu as pltpu
from jax.experimental.pallas import tpu_sc as plsc

N_NODES = 10000
N_EDGES = 320000
D_NODE = 128
D_EDGE = 16
D_OUT = 128

NC = 2
NS = 16
NW = NC * NS
EPT = N_EDGES // NW
CHUNK = 80
NCHUNK = EPT // CHUNK
ACC_W = 144
ROWS_PER_TILE = N_NODES // NS
LANES = 16

_HIGH = jax.lax.Precision.HIGHEST


def _proj_body(nf_ref, wn_ref, bn_ref, ef_ref, wm_ref,
               p_ref, q_ref, r_ref, e_ref):
    z = jnp.dot(nf_ref[...], wn_ref[...], precision=_HIGH) + bn_ref[...]
    p_ref[...] = z[:, 0:128]
    q_ref[...] = z[:, 128:256]
    r_ref[...] = z[:, 256:384]
    e_ref[...] = jnp.dot(ef_ref[...], wm_ref[...], precision=_HIGH)


def _tc_proj(nf, wn, bn, ef, wm):
    grid = 40
    nb = N_NODES // grid
    eb = N_EDGES // grid
    return pl.pallas_call(
        _proj_body,
        grid=(grid,),
        in_specs=[
            pl.BlockSpec((nb, D_NODE), lambda i: (i, 0)),
            pl.BlockSpec((D_NODE, 384), lambda i: (0, 0)),
            pl.BlockSpec((1, 384), lambda i: (0, 0)),
            pl.BlockSpec((eb, D_EDGE), lambda i: (i, 0)),
            pl.BlockSpec((D_EDGE, D_OUT), lambda i: (0, 0)),
        ],
        out_specs=[
            pl.BlockSpec((nb, D_OUT), lambda i: (i, 0)),
            pl.BlockSpec((nb, D_OUT), lambda i: (i, 0)),
            pl.BlockSpec((nb, D_OUT), lambda i: (i, 0)),
            pl.BlockSpec((eb, D_OUT), lambda i: (i, 0)),
        ],
        out_shape=[
            jax.ShapeDtypeStruct((N_NODES, D_OUT), jnp.float32),
            jax.ShapeDtypeStruct((N_NODES, D_OUT), jnp.float32),
            jax.ShapeDtypeStruct((N_NODES, D_OUT), jnp.float32),
            jax.ShapeDtypeStruct((N_EDGES, D_OUT), jnp.float32),
        ],
    )(nf, wn, bn, ef, wm)


def _sc_edge_body(p_hbm, q_hbm, e_hbm, src_hbm, dst_hbm, out_hbm,
                  acc, src_v, dst_v, p_v, q_v, e_v, m_v,
                  sem_p, sem_q, sem_e):
    c = jax.lax.axis_index("c")
    s = jax.lax.axis_index("s")
    wid = s * NC + c

    zero16 = jnp.zeros((LANES,), jnp.float32)
    lane = jax.lax.iota(jnp.int32, LANES)
    count_col = jnp.where(lane == 0, 1.0, 0.0).astype(jnp.float32)

    @pl.loop(0, CHUNK)
    def _(r):
        for seg in range(ACC_W // LANES):
            m_v[r, pl.ds(seg * LANES, LANES)] = zero16

    row0 = s * ROWS_PER_TILE
    nfull = ROWS_PER_TILE // CHUNK

    @pl.loop(0, nfull)
    def _(k):
        pltpu.sync_copy(m_v.at[pl.ds(0, CHUNK)],
                        acc.at[pl.ds(row0 + k * CHUNK, CHUNK)])

    rem = ROWS_PER_TILE % CHUNK
    if rem:
        pltpu.sync_copy(m_v.at[pl.ds(0, rem)],
                        acc.at[pl.ds(row0 + nfull * CHUNK, rem)])

    @pl.loop(0, CHUNK)
    def _(r):
        m_v[r, pl.ds(128, LANES)] = count_col

    plsc.subcore_barrier()

    @pl.loop(0, NCHUNK)
    def _(i):
        chunk_id = wid * NCHUNK + i
        ebase = wid * EPT + i * CHUNK
        pltpu.sync_copy(src_hbm.at[pl.ds(chunk_id, 1)], src_v)
        pltpu.sync_copy(dst_hbm.at[pl.ds(chunk_id, 1)], dst_v)
        cp_p = pltpu.async_copy(p_hbm.at[src_v.at[0]], p_v, sem_p)
        cp_q = pltpu.async_copy(q_hbm.at[dst_v.at[0]], q_v, sem_q)
        cp_e = pltpu.async_copy(e_hbm.at[pl.ds(ebase, CHUNK)], e_v, sem_e)
        cp_p.wait()
        cp_q.wait()
        cp_e.wait()

        @pl.loop(0, CHUNK)
        def _(r):
            for seg in range(D_OUT // LANES):
                sl = pl.ds(seg * LANES, LANES)
                m_v[r, sl] = jnp.maximum(p_v[r, sl] + q_v[r, sl] + e_v[r, sl],
                                         0.0)

        pltpu.sync_copy(m_v, acc.at[dst_v.at[0]], add=True)

    plsc.subcore_barrier()
    pltpu.sync_copy(acc.at[pl.ds(row0, ROWS_PER_TILE)],
                    out_hbm.at[c, pl.ds(row0, ROWS_PER_TILE)])


def _sc_edge(p, q, e, src, dst):
    mesh = plsc.VectorSubcoreMesh(core_axis_name="c", subcore_axis_name="s")
    kern = pl.kernel(
        _sc_edge_body,
        out_type=jax.ShapeDtypeStruct((NC, N_NODES, ACC_W), jnp.float32),
        mesh=mesh,
        scratch_types=[
            pltpu.VMEM_SHARED((N_NODES, ACC_W), jnp.float32),
            pltpu.VMEM((1, CHUNK), jnp.int32),
            pltpu.VMEM((1, CHUNK), jnp.int32),
            pltpu.VMEM((CHUNK, D_OUT), jnp.float32),
            pltpu.VMEM((CHUNK, D_OUT), jnp.float32),
            pltpu.VMEM((CHUNK, D_OUT), jnp.float32),
            pltpu.VMEM((CHUNK, ACC_W), jnp.float32),
            pltpu.SemaphoreType.DMA,
            pltpu.SemaphoreType.DMA,
            pltpu.SemaphoreType.DMA,
        ],
    )
    return kern(p, q, e, src, dst)


def _finish_body(parts_ref, r_ref, wv_ref, out_ref):
    s = parts_ref[0, :, 0:128] + parts_ref[1, :, 0:128]
    cnt = parts_ref[0, :, 128:129] + parts_ref[1, :, 128:129]
    neigh = s / jnp.maximum(cnt, 1.0)
    out_ref[...] = jax.nn.relu(
        jnp.dot(neigh, wv_ref[...], precision=_HIGH) + r_ref[...])


def _tc_finish(parts, r, wv):
    grid = 10
    nb = N_NODES // grid
    return pl.pallas_call(
        _finish_body,
        grid=(grid,),
        in_specs=[
            pl.BlockSpec((NC, nb, ACC_W), lambda i: (0, i, 0)),
            pl.BlockSpec((nb, D_OUT), lambda i: (i, 0)),
            pl.BlockSpec((D_OUT, D_OUT), lambda i: (0, 0)),
        ],
        out_specs=pl.BlockSpec((nb, D_OUT), lambda i: (i, 0)),
        out_shape=jax.ShapeDtypeStruct((N_NODES, D_OUT), jnp.float32),
    )(parts, r, wv)


def kernel(node_feats, edge_feats, edge_index, W_e, b_e, W_v, b_v):
    wn = jnp.concatenate([W_e[0:128], W_e[144:272], W_v[128:256]], axis=1)
    bn = jnp.concatenate(
        [b_e, jnp.zeros((128,), jnp.float32), b_v]).reshape(1, 384)
    wm = W_e[128:144]

    p, q, r, e = _tc_proj(node_feats, wn, bn, edge_feats, wm)

    src = edge_index[0].astype(jnp.int32).reshape(NW * NCHUNK, CHUNK)
    dst = edge_index[1].astype(jnp.int32).reshape(NW * NCHUNK, CHUNK)

    parts = _sc_edge(p, q, e, src, dst)
    return _tc_finish(parts, r, W_v[0:128])


# speedup vs baseline: 3.3483x; 3.3483x over previous
import dataclasses
import functools

import jax
import jax.numpy as jnp
from jax.experimental import pallas as pl
from jax.experimental.pallas import tpu as pltpu
from jax.experimental.pallas import tpu_sc as plsc

N_NODES = 10000
N_EDGES = 320000
D_NODE = 128
D_EDGE = 16
D_OUT = 128

NC = 2
NS = 16
NW = NC * NS
EPT = N_EDGES // NW
CHUNK = 80
NCHUNK = EPT // CHUNK
NROWCHUNK = N_NODES // CHUNK
CNT_ROWS = 80
LANES = 16

_HIGH = jax.lax.Precision.HIGHEST


def _proj_body(nf_ref, wn_ref, bn_ref, ef_ref, wm_ref,
               p_ref, q_ref, r_ref, e_ref):
    z = jnp.dot(nf_ref[...], wn_ref[...], precision=_HIGH) + bn_ref[...]
    p_ref[...] = z[:, 0:128]
    q_ref[...] = z[:, 128:256]
    r_ref[...] = z[:, 256:384]
    e_ref[...] = jnp.dot(ef_ref[...], wm_ref[...], precision=_HIGH)


def _tc_proj(nf, wn, bn, ef, wm):
    grid = 25
    nb = N_NODES // grid
    eb = N_EDGES // grid
    return pl.pallas_call(
        _proj_body,
        grid=(grid,),
        in_specs=[
            pl.BlockSpec((nb, D_NODE), lambda i: (i, 0)),
            pl.BlockSpec((D_NODE, 384), lambda i: (0, 0)),
            pl.BlockSpec((1, 384), lambda i: (0, 0)),
            pl.BlockSpec((eb, D_EDGE), lambda i: (i, 0)),
            pl.BlockSpec((D_EDGE, D_OUT), lambda i: (0, 0)),
        ],
        out_specs=[
            pl.BlockSpec((nb, D_OUT), lambda i: (i, 0)),
            pl.BlockSpec((nb, D_OUT), lambda i: (i, 0)),
            pl.BlockSpec((nb, D_OUT), lambda i: (i, 0)),
            pl.BlockSpec((eb, D_OUT), lambda i: (i, 0)),
        ],
        out_shape=[
            jax.ShapeDtypeStruct((N_NODES, D_OUT), jnp.float32),
            jax.ShapeDtypeStruct((N_NODES, D_OUT), jnp.float32),
            jax.ShapeDtypeStruct((N_NODES, D_OUT), jnp.float32),
            jax.ShapeDtypeStruct((N_EDGES, D_OUT), jnp.float32),
        ],
    )(nf, wn, bn, ef, wm)


def _sc_edge_body(p_hbm, q_hbm, e_hbm, src_hbm, dst_hbm,
                  out_s_hbm, out_c_hbm,
                  acc, acc_cnt, src_v, dst_v, cidx_v, p_v, q_v, m_v,
                  oh_v, sem_p, sem_q, sem_e):
    c = jax.lax.axis_index("c")
    s = jax.lax.axis_index("s")
    wid = s * NC + c

    zero16 = jnp.zeros((LANES,), jnp.float32)
    one16 = jnp.ones((LANES,), jnp.float32)
    iota16 = jax.lax.iota(jnp.int32, LANES)

    @pl.loop(0, CHUNK)
    def _(r):
        for seg in range(D_OUT // LANES):
            sl = pl.ds(seg * LANES, LANES)
            m_v[r, sl] = zero16
            oh_v[r, sl] = zero16

    @pl.loop(0, (NROWCHUNK + NS - 1) // NS)
    def _(j):
        k = j * NS + s

        @pl.when(k < NROWCHUNK)
        def _():
            pltpu.sync_copy(m_v.at[pl.ds(0, CHUNK)],
                            acc.at[pl.ds(k * CHUNK, CHUNK)])

    @pl.when(s == 0)
    def _():
        pltpu.sync_copy(m_v, acc_cnt)

    plsc.subcore_barrier()

    @pl.loop(0, NCHUNK)
    def _(i):
        ebase = wid * EPT + i * CHUNK
        pltpu.sync_copy(src_hbm.at[pl.ds(ebase, CHUNK)], src_v)
        pltpu.sync_copy(dst_hbm.at[pl.ds(ebase, CHUNK)], dst_v)
        cp_p = pltpu.async_copy(p_hbm.at[src_v], p_v, sem_p)
        cp_q = pltpu.async_copy(q_hbm.at[dst_v], q_v, sem_q)
        cp_e = pltpu.async_copy(e_hbm.at[pl.ds(ebase, CHUNK)], m_v, sem_e)

        for g in range(CHUNK // LANES):
            d16 = dst_v[pl.ds(g * LANES, LANES)]
            row16 = iota16 + (g * LANES)
            col16 = jax.lax.bitwise_and(d16, 127)
            cidx_v[pl.ds(g * LANES, LANES)] = jax.lax.shift_right_logical(
                d16, 7)
            plsc.store_scatter(oh_v, [row16, col16], one16)

        cp_p.wait()
        cp_q.wait()
        cp_e.wait()

        @pl.loop(0, CHUNK)
        def _(r):
            for seg in range(D_OUT // LANES):
                sl = pl.ds(seg * LANES, LANES)
                m_v[r, sl] = jnp.maximum(p_v[r, sl] + q_v[r, sl] + m_v[r, sl],
                                         0.0)

        pltpu.sync_copy(m_v, acc.at[dst_v], add=True)
        pltpu.sync_copy(oh_v, acc_cnt.at[cidx_v], add=True)

        for g in range(CHUNK // LANES):
            d16 = dst_v[pl.ds(g * LANES, LANES)]
            row16 = iota16 + (g * LANES)
            col16 = jax.lax.bitwise_and(d16, 127)
            plsc.store_scatter(oh_v, [row16, col16], zero16)

    plsc.subcore_barrier()

    @pl.loop(0, (NROWCHUNK + NS - 1) // NS)
    def _(j):
        k = j * NS + s

        @pl.when(k < NROWCHUNK)
        def _():
            pltpu.sync_copy(acc.at[pl.ds(k * CHUNK, CHUNK)],
                            out_s_hbm.at[c, pl.ds(k * CHUNK, CHUNK)])

    @pl.when(s == 0)
    def _():
        pltpu.sync_copy(acc_cnt, out_c_hbm.at[c])


def _sc_edge(p, q, e, src, dst):
    mesh = plsc.VectorSubcoreMesh(core_axis_name="c", subcore_axis_name="s")
    cp = pltpu.CompilerParams()
    if "needs_layout_passes" in pltpu.CompilerParams.__dataclass_fields__:
        cp = dataclasses.replace(cp, needs_layout_passes=False)
    kern = pl.kernel(
        _sc_edge_body,
        out_type=[
            jax.ShapeDtypeStruct((NC, N_NODES, D_OUT), jnp.float32),
            jax.ShapeDtypeStruct((NC, CNT_ROWS, 128), jnp.float32),
        ],
        mesh=mesh,
        scratch_types=[
            pltpu.VMEM_SHARED((N_NODES, D_OUT), jnp.float32),
            pltpu.VMEM_SHARED((CNT_ROWS, 128), jnp.float32),
            pltpu.VMEM((CHUNK,), jnp.int32),
            pltpu.VMEM((CHUNK,), jnp.int32),
            pltpu.VMEM((CHUNK,), jnp.int32),
            pltpu.VMEM((CHUNK, D_OUT), jnp.float32),
            pltpu.VMEM((CHUNK, D_OUT), jnp.float32),
            pltpu.VMEM((CHUNK, D_OUT), jnp.float32),
            pltpu.VMEM((CHUNK, D_OUT), jnp.float32),
            pltpu.SemaphoreType.DMA,
            pltpu.SemaphoreType.DMA,
            pltpu.SemaphoreType.DMA,
        ],
        compiler_params=cp,
    )
    return kern(p, q, e, src, dst)


def _finish_body(parts_ref, cnt_ref, r_ref, wv_ref, out_ref):
    s = parts_ref[0] + parts_ref[1]
    neigh = s / jnp.maximum(cnt_ref[...], 1.0)
    out_ref[...] = jax.nn.relu(
        jnp.dot(neigh, wv_ref[...], precision=_HIGH) + r_ref[...])


def _tc_finish(parts, cnt, r, wv):
    grid = 10
    nb = N_NODES // grid
    return pl.pallas_call(
        _finish_body,
        grid=(grid,),
        in_specs=[
            pl.BlockSpec((NC, nb, D_OUT), lambda i: (0, i, 0)),
            pl.BlockSpec((nb, 1), lambda i: (i, 0)),
            pl.BlockSpec((nb, D_OUT), lambda i: (i, 0)),
            pl.BlockSpec((D_OUT, D_OUT), lambda i: (0, 0)),
        ],
        out_specs=pl.BlockSpec((nb, D_OUT), lambda i: (i, 0)),
        out_shape=jax.ShapeDtypeStruct((N_NODES, D_OUT), jnp.float32),
    )(parts, cnt, r, wv)


def kernel(node_feats, edge_feats, edge_index, W_e, b_e, W_v, b_v):
    wn = jnp.concatenate([W_e[0:128], W_e[144:272], W_v[128:256]], axis=1)
    bn = jnp.concatenate(
        [b_e, jnp.zeros((128,), jnp.float32), b_v]).reshape(1, 384)
    wm = W_e[128:144]

    p, q, r, e = _tc_proj(node_feats, wn, bn, edge_feats, wm)

    src = edge_index[0].astype(jnp.int32)
    dst = edge_index[1].astype(jnp.int32)

    parts, parts_cnt = _sc_edge(p, q, e, src, dst)
    cnt = (parts_cnt[0] + parts_cnt[1]).reshape(-1)[:N_NODES]
    return _tc_finish(parts, cnt.reshape(N_NODES, 1), r, W_v[0:128])


# speedup vs baseline: 3.9069x; 1.1668x over previous
import dataclasses
import functools

import jax
import jax.numpy as jnp
from jax.experimental import pallas as pl
from jax.experimental.pallas import tpu as pltpu
from jax.experimental.pallas import tpu_sc as plsc

N_NODES = 10000
N_EDGES = 320000
D_NODE = 128
D_EDGE = 16
D_OUT = 128

NC = 2
NS = 16
NW = NC * NS
EPT = N_EDGES // NW
CHUNK = 40
NCHUNK = EPT // CHUNK
NROWCHUNK = N_NODES // CHUNK
CNT_ROWS = 80
LANES = 16

_HIGH = jax.lax.Precision.HIGHEST


def _proj_body(nf_ref, wn_ref, bn_ref, ef_ref, wm_ref,
               p_ref, q_ref, r_ref, e_ref):
    z = jnp.dot(nf_ref[...], wn_ref[...], precision=_HIGH) + bn_ref[...]
    p_ref[...] = z[:, 0:128]
    q_ref[...] = z[:, 128:256]
    r_ref[...] = z[:, 256:384]
    e_ref[...] = jnp.dot(ef_ref[...], wm_ref[...], precision=_HIGH)


def _tc_proj(nf, wn, bn, ef, wm):
    grid = 25
    nb = N_NODES // grid
    eb = N_EDGES // grid
    return pl.pallas_call(
        _proj_body,
        grid=(grid,),
        in_specs=[
            pl.BlockSpec((nb, D_NODE), lambda i: (i, 0)),
            pl.BlockSpec((D_NODE, 384), lambda i: (0, 0)),
            pl.BlockSpec((1, 384), lambda i: (0, 0)),
            pl.BlockSpec((eb, D_EDGE), lambda i: (i, 0)),
            pl.BlockSpec((D_EDGE, D_OUT), lambda i: (0, 0)),
        ],
        out_specs=[
            pl.BlockSpec((nb, D_OUT), lambda i: (i, 0)),
            pl.BlockSpec((nb, D_OUT), lambda i: (i, 0)),
            pl.BlockSpec((nb, D_OUT), lambda i: (i, 0)),
            pl.BlockSpec((eb, D_OUT), lambda i: (i, 0)),
        ],
        out_shape=[
            jax.ShapeDtypeStruct((N_NODES, D_OUT), jnp.float32),
            jax.ShapeDtypeStruct((N_NODES, D_OUT), jnp.float32),
            jax.ShapeDtypeStruct((N_NODES, D_OUT), jnp.float32),
            jax.ShapeDtypeStruct((N_EDGES, D_OUT), jnp.float32),
        ],
    )(nf, wn, bn, ef, wm)


def _sc_edge_body(p_hbm, q_hbm, e_hbm, src_hbm, dst_hbm,
                  out_s_hbm, out_c_hbm,
                  acc, acc_cnt,
                  src_a, dst_a, src_b, dst_b, cidx_v,
                  p_a, q_a, m_a, p_b, q_b, m_b, oh_v,
                  sem_sa, sem_da, sem_sb, sem_db,
                  sem_pa, sem_qa, sem_ea, sem_pb, sem_qb, sem_eb):
    c = jax.lax.axis_index("c")
    s = jax.lax.axis_index("s")
    wid = s * NC + c
    tbase = wid * EPT

    zero16 = jnp.zeros((LANES,), jnp.float32)
    one16 = jnp.ones((LANES,), jnp.float32)
    iota16 = jax.lax.iota(jnp.int32, LANES)
    mask_hi = iota16 >= 8

    @pl.loop(0, CHUNK)
    def _(r):
        for seg in range(D_OUT // LANES):
            sl = pl.ds(seg * LANES, LANES)
            m_a[r, sl] = zero16
            oh_v[r, sl] = zero16

    @pl.loop(0, (NROWCHUNK + NS - 1) // NS)
    def _(j):
        k = j * NS + s

        @pl.when(k < NROWCHUNK)
        def _():
            pltpu.sync_copy(m_a, acc.at[pl.ds(k * CHUNK, CHUNK)])

    @pl.when(s == 0)
    def _():
        pltpu.sync_copy(m_a, acc_cnt.at[pl.ds(0, CHUNK)])
        pltpu.sync_copy(m_a, acc_cnt.at[pl.ds(CHUNK, CHUNK)])

    plsc.subcore_barrier()

    def issue_idx(i, src_v, dst_v, sem_s, sem_d):
        eb = tbase + i * CHUNK
        pltpu.async_copy(src_hbm.at[pl.ds(eb, CHUNK)], src_v, sem_s)
        pltpu.async_copy(dst_hbm.at[pl.ds(eb, CHUNK)], dst_v, sem_d)

    def wait_idx(src_v, dst_v, sem_s, sem_d):
        pltpu.make_async_copy(src_hbm.at[pl.ds(0, CHUNK)], src_v, sem_s).wait()
        pltpu.make_async_copy(dst_hbm.at[pl.ds(0, CHUNK)], dst_v, sem_d).wait()

    def issue_gathers(i, src_v, dst_v, p_v, q_v, m_v, sem_p, sem_q, sem_e):
        eb = tbase + i * CHUNK
        pltpu.async_copy(p_hbm.at[src_v], p_v, sem_p)
        pltpu.async_copy(q_hbm.at[dst_v], q_v, sem_q)
        pltpu.async_copy(e_hbm.at[pl.ds(eb, CHUNK)], m_v, sem_e)

    def wait_gathers(src_v, dst_v, p_v, q_v, m_v, sem_p, sem_q, sem_e):
        pltpu.make_async_copy(p_hbm.at[src_v], p_v, sem_p).wait()
        pltpu.make_async_copy(q_hbm.at[dst_v], q_v, sem_q).wait()
        pltpu.make_async_copy(e_hbm.at[pl.ds(0, CHUNK)], m_v, sem_e).wait()

    def compute(p_v, q_v, m_v):
        @pl.loop(0, CHUNK)
        def _(r):
            for seg in range(D_OUT // LANES):
                sl = pl.ds(seg * LANES, LANES)
                m_v[r, sl] = jnp.maximum(
                    p_v[r, sl] + q_v[r, sl] + m_v[r, sl], 0.0)

    oh_groups = [(0, None), (16, None), (24, mask_hi)]

    def scatters(dst_v, m_v):
        pltpu.sync_copy(m_v, acc.at[dst_v], add=True)
        for off, msk in oh_groups:
            d16 = dst_v[pl.ds(off, LANES)]
            row16 = iota16 + off
            col16 = jax.lax.bitwise_and(d16, 127)
            cidx_v[pl.ds(off, LANES)] = jax.lax.shift_right_logical(d16, 7)
            plsc.store_scatter(oh_v, [row16, col16], one16, mask=msk)
        pltpu.sync_copy(oh_v, acc_cnt.at[cidx_v], add=True)
        for off, msk in oh_groups:
            d16 = dst_v[pl.ds(off, LANES)]
            row16 = iota16 + off
            col16 = jax.lax.bitwise_and(d16, 127)
            plsc.store_scatter(oh_v, [row16, col16], zero16, mask=msk)

    issue_idx(0, src_a, dst_a, sem_sa, sem_da)
    issue_idx(1, src_b, dst_b, sem_sb, sem_db)
    wait_idx(src_a, dst_a, sem_sa, sem_da)
    issue_gathers(0, src_a, dst_a, p_a, q_a, m_a, sem_pa, sem_qa, sem_ea)

    nbody = NCHUNK // 2

    @pl.loop(0, nbody)
    def _(j):
        i = j * 2
        wait_idx(src_b, dst_b, sem_sb, sem_db)
        issue_gathers(i + 1, src_b, dst_b, p_b, q_b, m_b,
                      sem_pb, sem_qb, sem_eb)
        wait_gathers(src_a, dst_a, p_a, q_a, m_a, sem_pa, sem_qa, sem_ea)
        compute(p_a, q_a, m_a)
        scatters(dst_a, m_a)

        @pl.when(j < nbody - 1)
        def _():
            issue_idx(i + 2, src_a, dst_a, sem_sa, sem_da)

        wait_gathers(src_b, dst_b, p_b, q_b, m_b, sem_pb, sem_qb, sem_eb)
        compute(p_b, q_b, m_b)
        scatters(dst_b, m_b)

        @pl.when(j < nbody - 1)
        def _():
            issue_idx(i + 3, src_b, dst_b, sem_sb, sem_db)
            wait_idx(src_a, dst_a, sem_sa, sem_da)
            issue_gathers(i + 2, src_a, dst_a, p_a, q_a, m_a,
                          sem_pa, sem_qa, sem_ea)

    plsc.subcore_barrier()

    @pl.loop(0, (NROWCHUNK + NS - 1) // NS)
    def _(j):
        k = j * NS + s

        @pl.when(k < NROWCHUNK)
        def _():
            pltpu.sync_copy(acc.at[pl.ds(k * CHUNK, CHUNK)],
                            out_s_hbm.at[c, pl.ds(k * CHUNK, CHUNK)])

    @pl.when(s == 0)
    def _():
        pltpu.sync_copy(acc_cnt, out_c_hbm.at[c])


def _sc_edge(p, q, e, src, dst):
    mesh = plsc.VectorSubcoreMesh(core_axis_name="c", subcore_axis_name="s")
    cp = pltpu.CompilerParams()
    if "needs_layout_passes" in pltpu.CompilerParams.__dataclass_fields__:
        cp = dataclasses.replace(cp, needs_layout_passes=False)
    kern = pl.kernel(
        _sc_edge_body,
        out_type=[
            jax.ShapeDtypeStruct((NC, N_NODES, D_OUT), jnp.float32),
            jax.ShapeDtypeStruct((NC, CNT_ROWS, 128), jnp.float32),
        ],
        mesh=mesh,
        scratch_types=[
            pltpu.VMEM_SHARED((N_NODES, D_OUT), jnp.float32),
            pltpu.VMEM_SHARED((CNT_ROWS, 128), jnp.float32),
            pltpu.VMEM((CHUNK,), jnp.int32),
            pltpu.VMEM((CHUNK,), jnp.int32),
            pltpu.VMEM((CHUNK,), jnp.int32),
            pltpu.VMEM((CHUNK,), jnp.int32),
            pltpu.VMEM((CHUNK,), jnp.int32),
            pltpu.VMEM((CHUNK, D_OUT), jnp.float32),
            pltpu.VMEM((CHUNK, D_OUT), jnp.float32),
            pltpu.VMEM((CHUNK, D_OUT), jnp.float32),
            pltpu.VMEM((CHUNK, D_OUT), jnp.float32),
            pltpu.VMEM((CHUNK, D_OUT), jnp.float32),
            pltpu.VMEM((CHUNK, D_OUT), jnp.float32),
            pltpu.VMEM((CHUNK, D_OUT), jnp.float32),
        ] + [pltpu.SemaphoreType.DMA] * 10,
        compiler_params=cp,
    )
    return kern(p, q, e, src, dst)


def _finish_body(parts_ref, cnt_ref, r_ref, wv_ref, out_ref):
    s = parts_ref[0] + parts_ref[1]
    neigh = s / jnp.maximum(cnt_ref[...], 1.0)
    out_ref[...] = jax.nn.relu(
        jnp.dot(neigh, wv_ref[...], precision=_HIGH) + r_ref[...])


def _tc_finish(parts, cnt, r, wv):
    grid = 10
    nb = N_NODES // grid
    return pl.pallas_call(
        _finish_body,
        grid=(grid,),
        in_specs=[
            pl.BlockSpec((NC, nb, D_OUT), lambda i: (0, i, 0)),
            pl.BlockSpec((nb, 1), lambda i: (i, 0)),
            pl.BlockSpec((nb, D_OUT), lambda i: (i, 0)),
            pl.BlockSpec((D_OUT, D_OUT), lambda i: (0, 0)),
        ],
        out_specs=pl.BlockSpec((nb, D_OUT), lambda i: (i, 0)),
        out_shape=jax.ShapeDtypeStruct((N_NODES, D_OUT), jnp.float32),
    )(parts, cnt, r, wv)


def kernel(node_feats, edge_feats, edge_index, W_e, b_e, W_v, b_v):
    wn = jnp.concatenate([W_e[0:128], W_e[144:272], W_v[128:256]], axis=1)
    bn = jnp.concatenate(
        [b_e, jnp.zeros((128,), jnp.float32), b_v]).reshape(1, 384)
    wm = W_e[128:144]

    p, q, r, e = _tc_proj(node_feats, wn, bn, edge_feats, wm)

    src = edge_index[0].astype(jnp.int32)
    dst = edge_index[1].astype(jnp.int32)

    parts, parts_cnt = _sc_edge(p, q, e, src, dst)
    cnt = (parts_cnt[0] + parts_cnt[1]).reshape(-1)[:N_NODES]
    return _tc_finish(parts, cnt.reshape(N_NODES, 1), r, W_v[0:128])
